# baseline (speedup 1.0000x reference)
import functools

import jax
import jax.numpy as jnp
from jax import lax
from jax.experimental import pallas as pl
from jax.experimental.pallas import tpu as pltpu
from jax.experimental.pallas import tpu_sc as plsc

_B = 4096
_L = 200
_D = 32
_H = 100
_C0 = 104
_C1 = _L - _C0
_NW = 32
_RPW = _B // _NW
_IPW = _RPW * _L
_NBUF = 8


def _sc_flatten_x(x):
    mesh = plsc.VectorSubcoreMesh(core_axis_name="c", subcore_axis_name="s")

    @functools.partial(
        pl.kernel,
        mesh=mesh,
        out_type=jax.ShapeDtypeStruct((_B * _L,), jnp.int32),
        compiler_params=pltpu.CompilerParams(use_tc_tiling_on_sc=True),
        scratch_types=[
            pltpu.VMEM((_RPW, _L), jnp.int32),
            pltpu.VMEM((_IPW,), jnp.int32),
        ],
    )
    def flatten(x_hbm, out_hbm, xin_v, xout_v):
        wid = lax.axis_index("s") * 2 + lax.axis_index("c")
        pltpu.sync_copy(x_hbm.at[pl.ds(wid * _RPW, _RPW)], xin_v)

        offs = tuple(range(0, _L - 16, 16)) + (_L - 16,)

        def row(r, carry):
            for c in offs:
                xout_v[pl.ds(r * _L + c, 16)] = xin_v[r, pl.ds(c, 16)]
            return carry

        lax.fori_loop(0, _RPW, row, 0)
        pltpu.sync_copy(xout_v, out_hbm.at[pl.ds(wid * _IPW, _IPW)])

    return flatten(x)


def _sc_pool(x_flat, table):
    mesh = plsc.VectorSubcoreMesh(core_axis_name="c", subcore_axis_name="s")

    @functools.partial(
        pl.kernel,
        mesh=mesh,
        out_type=jax.ShapeDtypeStruct((_B, _D), jnp.float32),
        compiler_params=pltpu.CompilerParams(use_tc_tiling_on_sc=False),
        scratch_types=[
            pltpu.VMEM((_IPW,), jnp.int32),
            pltpu.VMEM((_NBUF, _L, _D), jnp.float32),
            pltpu.VMEM((_RPW, _D), jnp.float32),
        ] + [pltpu.SemaphoreType.DMA] * _NBUF,
    )
    def pool(x_hbm, t_hbm, out_hbm, idx_v, bufs, out_v, *sems):
        wid = lax.axis_index("s") * 2 + lax.axis_index("c")
        pltpu.sync_copy(x_hbm.at[pl.ds(wid * _IPW, _IPW)], idx_v)

        def fire(r, slot):
            pltpu.async_copy(t_hbm.at[idx_v.at[pl.ds(r * _L, _C0)]],
                             bufs.at[slot, pl.ds(0, _C0)], sems[slot])
            pltpu.async_copy(t_hbm.at[idx_v.at[pl.ds(r * _L + _C0, _C1)]],
                             bufs.at[slot, pl.ds(_C0, _C1)], sems[slot])

        def drain(slot):
            pltpu.make_async_copy(t_hbm.at[pl.ds(0, _L)],
                                  bufs.at[slot], sems[slot]).wait()

        for j in range(_NBUF):
            fire(j, j)

        zero = jnp.zeros((16,), jnp.float32)

        def octet(k, carry):
            for slot in range(_NBUF):
                r = _NBUF * k + slot
                drain(slot)

                def acc(i, c, slot=slot):
                    a0, a1, a2, a3 = c
                    base = i * 20
                    for t in range(0, 20, 2):
                        a0 = a0 + bufs[slot, base + t, pl.ds(0, 16)]
                        a1 = a1 + bufs[slot, base + t, pl.ds(16, 16)]
                        a2 = a2 + bufs[slot, base + t + 1, pl.ds(0, 16)]
                        a3 = a3 + bufs[slot, base + t + 1, pl.ds(16, 16)]
                    return a0, a1, a2, a3

                a0, a1, a2, a3 = lax.fori_loop(
                    0, _L // 20, acc, (zero, zero, zero, zero))
                out_v[r, pl.ds(0, 16)] = a0 + a2
                out_v[r, pl.ds(16, 16)] = a1 + a3

                nr = r + _NBUF

                @pl.when(nr < _RPW)
                def _(nr=nr, slot=slot):
                    fire(nr, slot)
            return carry

        lax.fori_loop(0, _RPW // _NBUF, octet, 0)
        pltpu.sync_copy(out_v, out_hbm.at[pl.ds(wid * _RPW, _RPW)])

    return pool(x_flat, table)


def _tc_head(sums, lengths, W, b):
    def body(s_ref, l_ref, w_ref, b_ref, o_ref):
        acc = lax.dot_general(s_ref[...], w_ref[...],
                              (((1,), (1,)), ((), ())),
                              preferred_element_type=jnp.float32)
        inv = 1.0 / jnp.maximum(l_ref[...].astype(jnp.float32), 1.0)
        o_ref[...] = acc * inv + b_ref[...]

    return pl.pallas_call(
        body,
        out_shape=jax.ShapeDtypeStruct((_B, _H), jnp.float32),
    )(sums, lengths.reshape(_B, 1), W, b.reshape(1, _H))


def kernel(x, lengths, table, W, b):
    x = x.astype(jnp.int32)
    lengths = lengths.astype(jnp.int32)
    x_flat = _sc_flatten_x(x)
    sums = _sc_pool(x_flat, table)
    return _tc_head(sums, lengths, W, b)


# speedup vs baseline: 1.0051x; 1.0051x over previous
import functools

import jax
import jax.numpy as jnp
from jax import lax
from jax.experimental import pallas as pl
from jax.experimental.pallas import tpu as pltpu
from jax.experimental.pallas import tpu_sc as plsc

_B = 4096
_L = 200
_D = 32
_H = 100
_C0 = 104
_C1 = _L - _C0
_NW = 32
_RPW = _B // _NW
_IPW = _RPW * _L
_NBUF = 8


def _sc_pool(x, table):
    mesh = plsc.VectorSubcoreMesh(core_axis_name="c", subcore_axis_name="s")

    @functools.partial(
        pl.kernel,
        mesh=mesh,
        out_type=jax.ShapeDtypeStruct((_B, _D), jnp.float32),
        compiler_params=pltpu.CompilerParams(use_tc_tiling_on_sc=False),
        scratch_types=[
            pltpu.VMEM((_RPW, _L), jnp.int32),
            pltpu.VMEM((_NBUF, _L, _D), jnp.float32),
            pltpu.VMEM((_RPW, _D), jnp.float32),
        ] + [pltpu.SemaphoreType.DMA] * _NBUF,
    )
    def pool(x_hbm, t_hbm, out_hbm, idx_v, bufs, out_v, *sems):
        wid = lax.axis_index("s") * 2 + lax.axis_index("c")
        pltpu.sync_copy(x_hbm.at[pl.ds(wid * _RPW, _RPW)], idx_v)

        def fire(r, slot):
            pltpu.async_copy(t_hbm.at[idx_v.at[r, pl.ds(0, _C0)]],
                             bufs.at[slot, pl.ds(0, _C0)], sems[slot])
            pltpu.async_copy(t_hbm.at[idx_v.at[r, pl.ds(_C0, _C1)]],
                             bufs.at[slot, pl.ds(_C0, _C1)], sems[slot])

        def drain(slot):
            pltpu.make_async_copy(t_hbm.at[pl.ds(0, _L)],
                                  bufs.at[slot], sems[slot]).wait()

        for j in range(_NBUF):
            fire(j, j)

        zero = jnp.zeros((16,), jnp.float32)

        def octet(k, carry):
            for slot in range(_NBUF):
                r = _NBUF * k + slot
                drain(slot)

                def acc(i, c, slot=slot):
                    a0, a1, a2, a3 = c
                    base = i * 20
                    for t in range(0, 20, 2):
                        a0 = a0 + bufs[slot, base + t, pl.ds(0, 16)]
                        a1 = a1 + bufs[slot, base + t, pl.ds(16, 16)]
                        a2 = a2 + bufs[slot, base + t + 1, pl.ds(0, 16)]
                        a3 = a3 + bufs[slot, base + t + 1, pl.ds(16, 16)]
                    return a0, a1, a2, a3

                a0, a1, a2, a3 = lax.fori_loop(
                    0, _L // 20, acc, (zero, zero, zero, zero))
                out_v[r, pl.ds(0, 16)] = a0 + a2
                out_v[r, pl.ds(16, 16)] = a1 + a3

                nr = r + _NBUF

                @pl.when(nr < _RPW)
                def _(nr=nr, slot=slot):
                    fire(nr, slot)
            return carry

        lax.fori_loop(0, _RPW // _NBUF, octet, 0)
        pltpu.sync_copy(out_v, out_hbm.at[pl.ds(wid * _RPW, _RPW)])

    return pool(x, table)


def _tc_head(sums, lengths, W, b):
    def body(s_ref, l_ref, w_ref, b_ref, o_ref):
        acc = lax.dot_general(s_ref[...], w_ref[...],
                              (((1,), (1,)), ((), ())),
                              preferred_element_type=jnp.float32)
        inv = 1.0 / jnp.maximum(l_ref[...].astype(jnp.float32), 1.0)
        o_ref[...] = acc * inv + b_ref[...]

    return pl.pallas_call(
        body,
        out_shape=jax.ShapeDtypeStruct((_B, _H), jnp.float32),
    )(sums, lengths.reshape(_B, 1), W, b.reshape(1, _H))


def kernel(x, lengths, table, W, b):
    x = x.astype(jnp.int32)
    lengths = lengths.astype(jnp.int32)
    sums = _sc_pool(x, table)
    return _tc_head(sums, lengths, W, b)
